# Initial kernel scaffold; baseline (speedup 1.0000x reference)
import jax
import jax.numpy as jnp
from jax.experimental import pallas as pl


def kernel(x, Wq, bq, Wk, bk):
    raise NotImplementedError("write your pallas kernel here")


# speedup vs baseline: 1.0119x; 1.0119x over previous
import math

import jax
import jax.numpy as jnp
from jax.experimental import pallas as pl
from jax.experimental.pallas import tpu as pltpu

D = 2048
SEQ = 2048
B = 2
BM = 256
SCALE = 1.0 / math.sqrt(D)
BF = jnp.bfloat16


def _proj_body(x_ref, w_ref, b_ref, out_ref):
    x = x_ref[0].astype(BF)
    acc = jnp.dot(x, w_ref[...], preferred_element_type=jnp.float32)
    out_ref[0] = (acc + b_ref[...]).astype(BF)


def _scores_body(x_ref, wqt_ref, bq_ref, k_ref, aff_ref, pen_ref):
    x = x_ref[0].astype(BF)
    q = jnp.dot(x, wqt_ref[...], preferred_element_type=jnp.float32)
    q = (q + bq_ref[...]).astype(BF)
    s = jax.lax.dot_general(
        q, k_ref[0], (((1,), (1,)), ((), ())),
        preferred_element_type=jnp.float32)
    t = jnp.tanh(s * SCALE)
    aff_ref[0] = jnp.maximum(t, 0.0)
    pen_ref[0] = jnp.minimum(t, 0.0)


def kernel(x, Wq, bq, Wk, bk):
    wqt = Wq.T.astype(BF)
    wkt = Wk.T.astype(BF)
    bq2 = bq.reshape(1, D)
    bk2 = bk.reshape(1, D)

    grid = (B, SEQ // BM)

    k_bf = pl.pallas_call(
        _proj_body,
        grid=grid,
        in_specs=[
            pl.BlockSpec((1, BM, D), lambda b, i: (b, i, 0)),
            pl.BlockSpec((D, D), lambda b, i: (0, 0)),
            pl.BlockSpec((1, D), lambda b, i: (0, 0)),
        ],
        out_specs=pl.BlockSpec((1, BM, D), lambda b, i: (b, i, 0)),
        out_shape=jax.ShapeDtypeStruct((B, SEQ, D), BF),
        compiler_params=pltpu.CompilerParams(
            dimension_semantics=("parallel", "parallel")),
    )(x, wkt, bk2)

    aff, pen = pl.pallas_call(
        _scores_body,
        grid=grid,
        in_specs=[
            pl.BlockSpec((1, BM, D), lambda b, i: (b, i, 0)),
            pl.BlockSpec((D, D), lambda b, i: (0, 0)),
            pl.BlockSpec((1, D), lambda b, i: (0, 0)),
            pl.BlockSpec((1, SEQ, D), lambda b, i: (b, 0, 0)),
        ],
        out_specs=[
            pl.BlockSpec((1, BM, SEQ), lambda b, i: (b, i, 0)),
            pl.BlockSpec((1, BM, SEQ), lambda b, i: (b, i, 0)),
        ],
        out_shape=[
            jax.ShapeDtypeStruct((B, SEQ, SEQ), jnp.float32),
            jax.ShapeDtypeStruct((B, SEQ, SEQ), jnp.float32),
        ],
        compiler_params=pltpu.CompilerParams(
            dimension_semantics=("parallel", "parallel")),
    )(x, wqt, bq2, k_bf)

    return aff, pen


# speedup vs baseline: 1.0738x; 1.0612x over previous
import math

import jax
import jax.numpy as jnp
from jax.experimental import pallas as pl
from jax.experimental.pallas import tpu as pltpu

D = 2048
SEQ = 2048
B = 2
BM = 256
SCALE = 1.0 / math.sqrt(D)
BF = jnp.bfloat16


_NT = (((1,), (1,)), ((), ()))


def _proj_body(x_ref, w_ref, b_ref, out_ref):
    x = x_ref[0].astype(BF)
    acc = jax.lax.dot_general(x, w_ref[...], _NT,
                              preferred_element_type=jnp.float32)
    out_ref[0] = (acc + b_ref[...]).astype(BF)


def _scores_body(x_ref, wq_ref, bq_ref, k_ref, aff_ref, pen_ref):
    x = x_ref[0].astype(BF)
    q = jax.lax.dot_general(x, wq_ref[...], _NT,
                            preferred_element_type=jnp.float32)
    q = (q + bq_ref[...]).astype(BF)
    s = jax.lax.dot_general(q, k_ref[0], _NT,
                            preferred_element_type=jnp.float32)
    t = jnp.tanh(s * SCALE)
    aff_ref[0] = jnp.maximum(t, 0.0)
    pen_ref[0] = jnp.minimum(t, 0.0)


def kernel(x, Wq, bq, Wk, bk):
    wq_bf = Wq.astype(BF)
    wk_bf = Wk.astype(BF)
    bq2 = bq.reshape(1, D)
    bk2 = bk.reshape(1, D)

    grid = (B, SEQ // BM)

    k_bf = pl.pallas_call(
        _proj_body,
        grid=grid,
        in_specs=[
            pl.BlockSpec((1, BM, D), lambda b, i: (b, i, 0)),
            pl.BlockSpec((D, D), lambda b, i: (0, 0)),
            pl.BlockSpec((1, D), lambda b, i: (0, 0)),
        ],
        out_specs=pl.BlockSpec((1, BM, D), lambda b, i: (b, i, 0)),
        out_shape=jax.ShapeDtypeStruct((B, SEQ, D), BF),
        compiler_params=pltpu.CompilerParams(
            dimension_semantics=("parallel", "parallel")),
    )(x, wk_bf, bk2)

    aff, pen = pl.pallas_call(
        _scores_body,
        grid=grid,
        in_specs=[
            pl.BlockSpec((1, BM, D), lambda b, i: (b, i, 0)),
            pl.BlockSpec((D, D), lambda b, i: (0, 0)),
            pl.BlockSpec((1, D), lambda b, i: (0, 0)),
            pl.BlockSpec((1, SEQ, D), lambda b, i: (b, 0, 0)),
        ],
        out_specs=[
            pl.BlockSpec((1, BM, SEQ), lambda b, i: (b, i, 0)),
            pl.BlockSpec((1, BM, SEQ), lambda b, i: (b, i, 0)),
        ],
        out_shape=[
            jax.ShapeDtypeStruct((B, SEQ, SEQ), jnp.float32),
            jax.ShapeDtypeStruct((B, SEQ, SEQ), jnp.float32),
        ],
        compiler_params=pltpu.CompilerParams(
            dimension_semantics=("parallel", "parallel")),
    )(x, wq_bf, bq2, k_bf)

    return aff, pen


# speedup vs baseline: 1.1098x; 1.0335x over previous
import math

import jax
import jax.numpy as jnp
from jax.experimental import pallas as pl
from jax.experimental.pallas import tpu as pltpu

D = 2048
SEQ = 2048
B = 2
BM = 512
SCALE = 1.0 / math.sqrt(D)
BF = jnp.bfloat16


_NT = (((1,), (1,)), ((), ()))


def _proj_body(x_ref, w_ref, b_ref, out_ref):
    x = x_ref[0].astype(BF)
    acc = jax.lax.dot_general(x, w_ref[...], _NT,
                              preferred_element_type=jnp.float32)
    out_ref[0] = (acc + b_ref[...]).astype(BF)


def _scores_body(x_ref, wq_ref, bq_ref, k_ref, aff_ref, pen_ref):
    x = x_ref[0].astype(BF)
    q = jax.lax.dot_general(x, wq_ref[...], _NT,
                            preferred_element_type=jnp.float32)
    q = (q + bq_ref[...]).astype(BF)
    s = jax.lax.dot_general(q, k_ref[0], _NT,
                            preferred_element_type=jnp.float32)
    t = jnp.tanh(s * SCALE)
    aff_ref[0] = jnp.maximum(t, 0.0)
    pen_ref[0] = jnp.minimum(t, 0.0)


def kernel(x, Wq, bq, Wk, bk):
    wq_bf = Wq.astype(BF)
    wk_bf = Wk.astype(BF)
    bq2 = bq.reshape(1, D)
    bk2 = bk.reshape(1, D)

    grid = (B, SEQ // BM)

    k_bf = pl.pallas_call(
        _proj_body,
        grid=grid,
        in_specs=[
            pl.BlockSpec((1, BM, D), lambda b, i: (b, i, 0)),
            pl.BlockSpec((D, D), lambda b, i: (0, 0)),
            pl.BlockSpec((1, D), lambda b, i: (0, 0)),
        ],
        out_specs=pl.BlockSpec((1, BM, D), lambda b, i: (b, i, 0)),
        out_shape=jax.ShapeDtypeStruct((B, SEQ, D), BF),
        compiler_params=pltpu.CompilerParams(
            dimension_semantics=("parallel", "parallel")),
    )(x, wk_bf, bk2)

    aff, pen = pl.pallas_call(
        _scores_body,
        grid=grid,
        in_specs=[
            pl.BlockSpec((1, BM, D), lambda b, i: (b, i, 0)),
            pl.BlockSpec((D, D), lambda b, i: (0, 0)),
            pl.BlockSpec((1, D), lambda b, i: (0, 0)),
            pl.BlockSpec((1, SEQ, D), lambda b, i: (b, 0, 0)),
        ],
        out_specs=[
            pl.BlockSpec((1, BM, SEQ), lambda b, i: (b, i, 0)),
            pl.BlockSpec((1, BM, SEQ), lambda b, i: (b, i, 0)),
        ],
        out_shape=[
            jax.ShapeDtypeStruct((B, SEQ, SEQ), jnp.float32),
            jax.ShapeDtypeStruct((B, SEQ, SEQ), jnp.float32),
        ],
        compiler_params=pltpu.CompilerParams(
            dimension_semantics=("parallel", "parallel")),
    )(x, wq_bf, bq2, k_bf)

    return aff, pen
